# baseline (speedup 1.0000x reference)
import functools

import jax
import jax.numpy as jnp
import numpy as np
from jax.experimental import pallas as pl

B, T, N, CIN, HID, NL, NH = 16, 32, 116, 116, 128, 4, 8
G = B * T
E = N * N
KTH = 9419
TG = 64

_SIGN = np.int32(-2**31)
_MAGM = np.int32(0x7FFFFFFF)


def _sortable_key(bits):
    return jnp.where(bits >= 0, bits, bits ^ _MAGM)


def _thr_kernel(a_ref, thr_ref):
    bits = jax.lax.bitcast_convert_type(a_ref[...], jnp.int32)
    key = _sortable_key(bits)
    ans = jnp.full((TG, 1), _SIGN, jnp.int32)
    for bit in range(31, -1, -1):
        m = _SIGN if bit == 31 else np.int32(1 << bit)
        cand = ans ^ m
        c = jnp.sum((key < cand).astype(jnp.int32), axis=1, keepdims=True)
        ans = jnp.where(c < KTH, cand, ans)
    fb = jnp.where(ans >= 0, ans, ans ^ _MAGM)
    thr_ref[...] = jax.lax.bitcast_convert_type(fb, jnp.float32)


GB = 8


def _graph_kernel(thr_ref, v_ref, a_ref, Wi_ref, bi_ref, epsv_ref,
                  gW1_ref, s1_ref, t1_ref, gW2_ref, s2_ref, t2_ref,
                  sWe_ref, ses_ref, set_ref, sWa_ref, sba_ref,
                  hro_ref, og_ref):
    f32 = jnp.float32
    col = jax.lax.broadcasted_iota(jnp.int32, (N, N), 1)
    row = jax.lax.broadcasted_iota(jnp.int32, (N, N), 0)
    rs = range(GB)
    A = [(a_ref[g] > thr_ref[g, 0, 0]).astype(f32) for g in rs]
    x = [jax.lax.dot_general(v_ref[g], Wi_ref[...], (((1,), (1,)), ((), ())),
                             preferred_element_type=f32) + bi_ref[...]
         for g in rs]
    hros = [[] for _ in rs]
    ogs = [[] for _ in rs]
    for l in range(NL):
        epsb = epsv_ref[l:l + 1, :]
        s1 = s1_ref[l:l + 1, :]
        t1 = t1_ref[l:l + 1, :]
        s2 = s2_ref[l:l + 1, :]
        t2 = t2_ref[l:l + 1, :]
        agg = [jnp.dot(A[g], x[g], preferred_element_type=f32) + epsb * x[g]
               for g in rs]
        h1 = [jax.lax.dot_general(agg[g], gW1_ref[l], (((1,), (1,)), ((), ())),
                                  preferred_element_type=f32) for g in rs]
        z = [jax.nn.relu(h1[g] * s1 + t1) for g in rs]
        h2 = [jax.lax.dot_general(z[g], gW2_ref[l], (((1,), (1,)), ((), ())),
                                  preferred_element_type=f32) for g in rs]
        x = [jax.nn.relu(h2[g] * s2 + t2) for g in rs]
        xr = [jnp.mean(x[g], axis=0, keepdims=True) for g in rs]
        e = [jax.lax.dot_general(xr[g], sWe_ref[l], (((1,), (1,)), ((), ())),
                                 preferred_element_type=f32) for g in rs]
        ep = [e[g] * ses_ref[l:l + 1, :] + set_ref[l:l + 1, :] for g in rs]
        xe = [0.5 * ep[g] * (1.0 + jax.lax.erf(
            ep[g] * np.float32(1.0 / np.sqrt(2.0)))) for g in rs]
        gp = [jax.lax.dot_general(xe[g], sWa_ref[l], (((1,), (1,)), ((), ())),
                                  preferred_element_type=f32)
              + sba_ref[l:l + 1, :] for g in rs]
        ga = [jax.nn.sigmoid(gp[g]) for g in rs]
        for g in rs:
            hro = jnp.dot(ga[g], x[g], preferred_element_type=f32) \
                * np.float32(1.0 / N)
            hros[g].append(hro.reshape(1, 1, 1, HID))
        Mi = [jax.lax.dot_general(x[g], x[g], (((1,), (1,)), ((), ())),
                                  preferred_element_type=f32) for g in rs]
        for g in rs:
            rmax = jnp.max(Mi[g], axis=1, keepdims=True)
            Dn = Mi[g] / rmax
            D = jnp.where(col >= row, Dn - jnp.where(col == row, 1.0, 0.0), 0.0)
            ogs[g].append(jnp.sqrt(jnp.sum(D * D)).reshape(1, 1, 1))
    hro_ref[...] = jnp.concatenate(
        [jnp.concatenate(hros[g], axis=2) for g in rs], axis=1)
    og_ref[...] = jnp.concatenate(
        [jnp.concatenate(ogs[g], axis=2) for g in rs], axis=0)


def _ln(z, g, b):
    mu = jnp.mean(z, axis=-1, keepdims=True)
    var = jnp.mean((z - mu) ** 2, axis=-1, keepdims=True)
    return (z - mu) * jax.lax.rsqrt(var + 1e-5) * g + b


def _tx_kernel(h_ref, og_ref, tWin_ref, tbin_ref, tWout_ref, tbout_ref,
               ln1g_ref, ln1b_ref, ln2g_ref, ln2b_ref,
               mW1_ref, mb1_ref, mW2_ref, mb2_ref,
               lat_ref, reg_ref):
    f32 = jnp.float32
    DH = HID // NH
    lane_head = jax.lax.broadcasted_iota(jnp.int32, (1, HID), 1) // DH
    head_masks = [(lane_head == h).astype(f32) for h in range(NH)]
    lats = []
    for l in range(NL):
        h2 = h_ref[:, :, l, :].reshape(B * T, HID)
        qkv = jax.lax.dot_general(h2, tWin_ref[l], (((1,), (1,)), ((), ())),
                                  preferred_element_type=f32) + tbin_ref[l]
        q = qkv[:, :HID] * np.float32(1.0 / np.sqrt(DH))
        k = qkv[:, HID:2 * HID]
        v = qkv[:, 2 * HID:]
        obs = []
        for b in range(B):
            qb = q[b * T:(b + 1) * T, :]
            kb = k[b * T:(b + 1) * T, :]
            vb = v[b * T:(b + 1) * T, :]
            ob = jnp.zeros((T, HID), f32)
            for h in range(NH):
                mh = head_masks[h]
                s = jax.lax.dot_general(qb * mh, kb, (((1,), (1,)), ((), ())),
                                        preferred_element_type=f32)
                p = jax.nn.softmax(s, axis=-1)
                ob = ob + jnp.dot(p, vb * mh, preferred_element_type=f32)
            obs.append(ob)
        o3 = jnp.concatenate(obs, axis=0)
        o = jax.lax.dot_general(o3, tWout_ref[l], (((1,), (1,)), ((), ())),
                                preferred_element_type=f32) + tbout_ref[l]
        xat = _ln(o, ln1g_ref[l], ln1b_ref[l])
        m1 = jax.nn.relu(
            jax.lax.dot_general(xat, mW1_ref[l], (((1,), (1,)), ((), ())),
                                preferred_element_type=f32) + mb1_ref[l])
        m2 = jax.lax.dot_general(m1, mW2_ref[l], (((1,), (1,)), ((), ())),
                                 preferred_element_type=f32) + mb2_ref[l]
        xat = _ln(xat + m2, ln2g_ref[l], ln2b_ref[l])
        lats.append(jnp.sum(xat.reshape(B, T, HID), axis=1))
    lat_ref[...] = jnp.concatenate(lats, axis=1)
    reg_ref[...] = (jnp.sum(og_ref[...]) * np.float32(1.0 / G)).reshape(1, 1)


def _full(shape):
    nd = len(shape)
    return pl.BlockSpec(shape, lambda i, _nd=nd: (0,) * _nd)


@jax.jit
def kernel(v1, a1, t1, sampling_endpoints1, W_init, b_init, eps,
           gW1, gb1, gW2, gb2,
           gbn1_g, gbn1_b, gbn1_rm, gbn1_rv,
           gbn2_g, gbn2_b, gbn2_rm, gbn2_rv,
           sbn_g, sbn_b, sbn_rm, sbn_rv,
           sW_e, sb_e, sW_a, sb_a,
           tWin, tbin, tWout, tbout,
           ln1_g, ln1_b, ln2_g, ln2_b,
           mW1, mb1, mW2, mb2):
    f32 = jnp.float32
    a_flat = a1.reshape(G, E)
    thr = pl.pallas_call(
        _thr_kernel,
        grid=(G // TG,),
        in_specs=[pl.BlockSpec((TG, E), lambda i: (i, 0))],
        out_specs=pl.BlockSpec((TG, 1), lambda i: (i, 0)),
        out_shape=jax.ShapeDtypeStruct((G, 1), f32),
    )(a_flat)

    def fold(bias, g, b, rm, rv):
        sc = g * jax.lax.rsqrt(rv + 1e-5)
        return sc, (bias - rm) * sc + b

    s1, t1f = fold(gb1, gbn1_g, gbn1_b, gbn1_rm, gbn1_rv)
    s2, t2f = fold(gb2, gbn2_g, gbn2_b, gbn2_rm, gbn2_rv)
    ses, setf = fold(sb_e, sbn_g, sbn_b, sbn_rm, sbn_rv)
    epsv = jnp.broadcast_to(eps.reshape(NL, 1), (NL, HID))

    thr3 = thr.reshape(G, 1, 1)
    v3 = v1.reshape(G, N, CIN)
    a3 = a1.reshape(G, N, N)
    hro, og = pl.pallas_call(
        _graph_kernel,
        grid=(G // GB,),
        in_specs=[
            pl.BlockSpec((GB, 1, 1), lambda i: (i, 0, 0)),
            pl.BlockSpec((GB, N, CIN), lambda i: (i, 0, 0)),
            pl.BlockSpec((GB, N, N), lambda i: (i, 0, 0)),
            _full((HID, CIN)), _full((1, HID)), _full((NL, HID)),
            _full((NL, HID, HID)), _full((NL, HID)), _full((NL, HID)),
            _full((NL, HID, HID)), _full((NL, HID)), _full((NL, HID)),
            _full((NL, HID, HID)), _full((NL, HID)), _full((NL, HID)),
            _full((NL, N, HID)), _full((NL, N)),
        ],
        out_specs=[
            pl.BlockSpec((1, GB, NL, HID), lambda i: (i // (T // GB), i % (T // GB), 0, 0)),
            pl.BlockSpec((GB, 1, NL), lambda i: (i, 0, 0)),
        ],
        out_shape=[
            jax.ShapeDtypeStruct((B, T, NL, HID), f32),
            jax.ShapeDtypeStruct((G, 1, NL), f32),
        ],
    )(thr3, v3, a3, W_init, b_init.reshape(1, HID), epsv,
      gW1, s1, t1f, gW2, s2, t2f, sW_e, ses, setf, sW_a, sb_a)

    lat, reg2 = pl.pallas_call(
        _tx_kernel,
        grid=(1,),
        in_specs=[
            _full((B, T, NL, HID)),
            _full((G, 1, NL)),
            _full((NL, 3 * HID, HID)),
            _full((NL, 1, 3 * HID)),
            _full((NL, HID, HID)),
            _full((NL, 1, HID)),
            _full((NL, 1, HID)),
            _full((NL, 1, HID)),
            _full((NL, 1, HID)),
            _full((NL, 1, HID)),
            _full((NL, 2 * HID, HID)),
            _full((NL, 1, 2 * HID)),
            _full((NL, HID, 2 * HID)),
            _full((NL, 1, HID)),
        ],
        out_specs=[
            pl.BlockSpec((B, NL * HID), lambda i: (0, 0)),
            pl.BlockSpec((1, 1), lambda i: (0, 0)),
        ],
        out_shape=[
            jax.ShapeDtypeStruct((B, NL * HID), f32),
            jax.ShapeDtypeStruct((1, 1), f32),
        ],
    )(hro, og, tWin, tbin.reshape(NL, 1, 3 * HID), tWout,
      tbout.reshape(NL, 1, HID),
      ln1_g.reshape(NL, 1, HID), ln1_b.reshape(NL, 1, HID),
      ln2_g.reshape(NL, 1, HID), ln2_b.reshape(NL, 1, HID),
      mW1, mb1.reshape(NL, 1, 2 * HID), mW2, mb2.reshape(NL, 1, HID))

    reg = reg2.reshape(())
    return lat, reg


# speedup vs baseline: 1.1026x; 1.1026x over previous
import functools

import jax
import jax.numpy as jnp
import numpy as np
from jax.experimental import pallas as pl
from jax.experimental.pallas import tpu as pltpu

B, T, N, CIN, HID, NL, NH = 16, 32, 116, 116, 128, 4, 8
G = B * T
E = N * N
KTH = 9419
TG = 64

_SIGN = np.int32(-2**31)
_MAGM = np.int32(0x7FFFFFFF)


def _sortable_key(bits):
    return jnp.where(bits >= 0, bits, bits ^ _MAGM)


def _thr_kernel(a_ref, thr_ref):
    bits = jax.lax.bitcast_convert_type(a_ref[...], jnp.int32)
    key = _sortable_key(bits)
    ans = jnp.full((TG, 1), _SIGN, jnp.int32)
    for bit in range(31, -1, -1):
        m = _SIGN if bit == 31 else np.int32(1 << bit)
        cand = ans ^ m
        c = jnp.sum((key < cand).astype(jnp.int32), axis=1, keepdims=True)
        ans = jnp.where(c < KTH, cand, ans)
    fb = jnp.where(ans >= 0, ans, ans ^ _MAGM)
    thr_ref[...] = jax.lax.bitcast_convert_type(fb, jnp.float32)


GB = 8


def _graph_kernel(thr_ref, v_ref, a_ref, Wi_ref, bi_ref, epsv_ref,
                  gW1_ref, s1_ref, t1_ref, gW2_ref, s2_ref, t2_ref,
                  sWe_ref, ses_ref, set_ref, sWa_ref, sba_ref,
                  hro_ref, og_ref):
    f32 = jnp.float32
    col = jax.lax.broadcasted_iota(jnp.int32, (N, N), 1)
    row = jax.lax.broadcasted_iota(jnp.int32, (N, N), 0)
    rs = range(GB)
    A = [(a_ref[g] > thr_ref[g, 0, 0]).astype(f32) for g in rs]
    x = [jax.lax.dot_general(v_ref[g], Wi_ref[...], (((1,), (1,)), ((), ())),
                             preferred_element_type=f32) + bi_ref[...]
         for g in rs]
    hros = [[] for _ in rs]
    ogs = [[] for _ in rs]
    for l in range(NL):
        epsb = epsv_ref[l:l + 1, :]
        s1 = s1_ref[l:l + 1, :]
        t1 = t1_ref[l:l + 1, :]
        s2 = s2_ref[l:l + 1, :]
        t2 = t2_ref[l:l + 1, :]
        agg = [jnp.dot(A[g], x[g], preferred_element_type=f32) + epsb * x[g]
               for g in rs]
        h1 = [jax.lax.dot_general(agg[g], gW1_ref[l], (((1,), (1,)), ((), ())),
                                  preferred_element_type=f32) for g in rs]
        z = [jax.nn.relu(h1[g] * s1 + t1) for g in rs]
        h2 = [jax.lax.dot_general(z[g], gW2_ref[l], (((1,), (1,)), ((), ())),
                                  preferred_element_type=f32) for g in rs]
        x = [jax.nn.relu(h2[g] * s2 + t2) for g in rs]
        xr = [jnp.mean(x[g], axis=0, keepdims=True) for g in rs]
        e = [jax.lax.dot_general(xr[g], sWe_ref[l], (((1,), (1,)), ((), ())),
                                 preferred_element_type=f32) for g in rs]
        ep = [e[g] * ses_ref[l:l + 1, :] + set_ref[l:l + 1, :] for g in rs]
        xe = [0.5 * ep[g] * (1.0 + jax.lax.erf(
            ep[g] * np.float32(1.0 / np.sqrt(2.0)))) for g in rs]
        gp = [jax.lax.dot_general(xe[g], sWa_ref[l], (((1,), (1,)), ((), ())),
                                  preferred_element_type=f32)
              + sba_ref[l:l + 1, :] for g in rs]
        ga = [jax.nn.sigmoid(gp[g]) for g in rs]
        for g in rs:
            hro = jnp.dot(ga[g], x[g], preferred_element_type=f32) \
                * np.float32(1.0 / N)
            hros[g].append(hro.reshape(1, 1, 1, HID))
        Mi = [jax.lax.dot_general(x[g], x[g], (((1,), (1,)), ((), ())),
                                  preferred_element_type=f32) for g in rs]
        for g in rs:
            rmax = jnp.max(Mi[g], axis=1, keepdims=True)
            Dn = Mi[g] / rmax
            D = jnp.where(col >= row, Dn - jnp.where(col == row, 1.0, 0.0), 0.0)
            ogs[g].append(jnp.sqrt(jnp.sum(D * D)).reshape(1, 1, 1))
    hro_ref[...] = jnp.concatenate(
        [jnp.concatenate(hros[g], axis=2) for g in rs], axis=1)
    og_ref[...] = jnp.concatenate(
        [jnp.concatenate(ogs[g], axis=2) for g in rs], axis=0)


def _ln(z, g, b):
    mu = jnp.mean(z, axis=-1, keepdims=True)
    var = jnp.mean((z - mu) ** 2, axis=-1, keepdims=True)
    return (z - mu) * jax.lax.rsqrt(var + 1e-5) * g + b


def _tx_kernel(h_ref, og_ref, tWin_ref, tbin_ref, tWout_ref, tbout_ref,
               ln1g_ref, ln1b_ref, ln2g_ref, ln2b_ref,
               mW1_ref, mb1_ref, mW2_ref, mb2_ref,
               lat_ref, reg_ref):
    f32 = jnp.float32
    DH = HID // NH
    lane_head = jax.lax.broadcasted_iota(jnp.int32, (1, HID), 1) // DH
    head_masks = [(lane_head == h).astype(f32) for h in range(NH)]
    lats = []
    for l in range(NL):
        h2 = h_ref[:, :, l, :].reshape(B * T, HID)
        qkv = jax.lax.dot_general(h2, tWin_ref[l], (((1,), (1,)), ((), ())),
                                  preferred_element_type=f32) + tbin_ref[l]
        q = qkv[:, :HID] * np.float32(1.0 / np.sqrt(DH))
        k = qkv[:, HID:2 * HID]
        v = qkv[:, 2 * HID:]
        obs = []
        for b in range(B):
            qb = q[b * T:(b + 1) * T, :]
            kb = k[b * T:(b + 1) * T, :]
            vb = v[b * T:(b + 1) * T, :]
            ob = jnp.zeros((T, HID), f32)
            for h in range(NH):
                mh = head_masks[h]
                s = jax.lax.dot_general(qb * mh, kb, (((1,), (1,)), ((), ())),
                                        preferred_element_type=f32)
                p = jax.nn.softmax(s, axis=-1)
                ob = ob + jnp.dot(p, vb * mh, preferred_element_type=f32)
            obs.append(ob)
        o3 = jnp.concatenate(obs, axis=0)
        o = jax.lax.dot_general(o3, tWout_ref[l], (((1,), (1,)), ((), ())),
                                preferred_element_type=f32) + tbout_ref[l]
        xat = _ln(o, ln1g_ref[l], ln1b_ref[l])
        m1 = jax.nn.relu(
            jax.lax.dot_general(xat, mW1_ref[l], (((1,), (1,)), ((), ())),
                                preferred_element_type=f32) + mb1_ref[l])
        m2 = jax.lax.dot_general(m1, mW2_ref[l], (((1,), (1,)), ((), ())),
                                 preferred_element_type=f32) + mb2_ref[l]
        xat = _ln(xat + m2, ln2g_ref[l], ln2b_ref[l])
        lats.append(jnp.sum(xat.reshape(B, T, HID), axis=1))
    lat_ref[...] = jnp.concatenate(lats, axis=1)
    reg_ref[...] = (jnp.sum(og_ref[...]) * np.float32(1.0 / G)).reshape(1, 1)


def _full(shape):
    nd = len(shape)
    return pl.BlockSpec(shape, lambda i, _nd=nd: (0,) * _nd)


@jax.jit
def kernel(v1, a1, t1, sampling_endpoints1, W_init, b_init, eps,
           gW1, gb1, gW2, gb2,
           gbn1_g, gbn1_b, gbn1_rm, gbn1_rv,
           gbn2_g, gbn2_b, gbn2_rm, gbn2_rv,
           sbn_g, sbn_b, sbn_rm, sbn_rv,
           sW_e, sb_e, sW_a, sb_a,
           tWin, tbin, tWout, tbout,
           ln1_g, ln1_b, ln2_g, ln2_b,
           mW1, mb1, mW2, mb2):
    f32 = jnp.float32
    a_flat = a1.reshape(G, E)
    thr = pl.pallas_call(
        _thr_kernel,
        grid=(G // TG,),
        compiler_params=pltpu.CompilerParams(
            dimension_semantics=("parallel",)),
        in_specs=[pl.BlockSpec((TG, E), lambda i: (i, 0))],
        out_specs=pl.BlockSpec((TG, 1), lambda i: (i, 0)),
        out_shape=jax.ShapeDtypeStruct((G, 1), f32),
    )(a_flat)

    def fold(bias, g, b, rm, rv):
        sc = g * jax.lax.rsqrt(rv + 1e-5)
        return sc, (bias - rm) * sc + b

    s1, t1f = fold(gb1, gbn1_g, gbn1_b, gbn1_rm, gbn1_rv)
    s2, t2f = fold(gb2, gbn2_g, gbn2_b, gbn2_rm, gbn2_rv)
    ses, setf = fold(sb_e, sbn_g, sbn_b, sbn_rm, sbn_rv)
    epsv = jnp.broadcast_to(eps.reshape(NL, 1), (NL, HID))

    thr3 = thr.reshape(G, 1, 1)
    v3 = v1.reshape(G, N, CIN)
    a3 = a1.reshape(G, N, N)
    hro, og = pl.pallas_call(
        _graph_kernel,
        grid=(G // GB,),
        compiler_params=pltpu.CompilerParams(
            dimension_semantics=("parallel",)),
        in_specs=[
            pl.BlockSpec((GB, 1, 1), lambda i: (i, 0, 0)),
            pl.BlockSpec((GB, N, CIN), lambda i: (i, 0, 0)),
            pl.BlockSpec((GB, N, N), lambda i: (i, 0, 0)),
            _full((HID, CIN)), _full((1, HID)), _full((NL, HID)),
            _full((NL, HID, HID)), _full((NL, HID)), _full((NL, HID)),
            _full((NL, HID, HID)), _full((NL, HID)), _full((NL, HID)),
            _full((NL, HID, HID)), _full((NL, HID)), _full((NL, HID)),
            _full((NL, N, HID)), _full((NL, N)),
        ],
        out_specs=[
            pl.BlockSpec((1, GB, NL, HID), lambda i: (i // (T // GB), i % (T // GB), 0, 0)),
            pl.BlockSpec((GB, 1, NL), lambda i: (i, 0, 0)),
        ],
        out_shape=[
            jax.ShapeDtypeStruct((B, T, NL, HID), f32),
            jax.ShapeDtypeStruct((G, 1, NL), f32),
        ],
    )(thr3, v3, a3, W_init, b_init.reshape(1, HID), epsv,
      gW1, s1, t1f, gW2, s2, t2f, sW_e, ses, setf, sW_a, sb_a)

    lat, reg2 = pl.pallas_call(
        _tx_kernel,
        grid=(1,),
        in_specs=[
            _full((B, T, NL, HID)),
            _full((G, 1, NL)),
            _full((NL, 3 * HID, HID)),
            _full((NL, 1, 3 * HID)),
            _full((NL, HID, HID)),
            _full((NL, 1, HID)),
            _full((NL, 1, HID)),
            _full((NL, 1, HID)),
            _full((NL, 1, HID)),
            _full((NL, 1, HID)),
            _full((NL, 2 * HID, HID)),
            _full((NL, 1, 2 * HID)),
            _full((NL, HID, 2 * HID)),
            _full((NL, 1, HID)),
        ],
        out_specs=[
            pl.BlockSpec((B, NL * HID), lambda i: (0, 0)),
            pl.BlockSpec((1, 1), lambda i: (0, 0)),
        ],
        out_shape=[
            jax.ShapeDtypeStruct((B, NL * HID), f32),
            jax.ShapeDtypeStruct((1, 1), f32),
        ],
    )(hro, og, tWin, tbin.reshape(NL, 1, 3 * HID), tWout,
      tbout.reshape(NL, 1, HID),
      ln1_g.reshape(NL, 1, HID), ln1_b.reshape(NL, 1, HID),
      ln2_g.reshape(NL, 1, HID), ln2_b.reshape(NL, 1, HID),
      mW1, mb1.reshape(NL, 1, 2 * HID), mW2, mb2.reshape(NL, 1, HID))

    reg = reg2.reshape(())
    return lat, reg


# speedup vs baseline: 1.1367x; 1.0309x over previous
import functools

import jax
import jax.numpy as jnp
import numpy as np
from jax.experimental import pallas as pl
from jax.experimental.pallas import tpu as pltpu

B, T, N, CIN, HID, NL, NH = 16, 32, 116, 116, 128, 4, 8
G = B * T
E = N * N
KTH = 9419
TG = 64

_SIGN = np.int32(-2**31)
_MAGM = np.int32(0x7FFFFFFF)


def _sortable_key(bits):
    return jnp.where(bits >= 0, bits, bits ^ _MAGM)


def _thr_kernel(a_ref, thr_ref):
    bits = jax.lax.bitcast_convert_type(a_ref[...], jnp.int32)
    key = _sortable_key(bits)
    ans = jnp.full((TG, 1), _SIGN, jnp.int32)
    for bit in range(31, -1, -1):
        m = _SIGN if bit == 31 else np.int32(1 << bit)
        cand = ans ^ m
        c = jnp.sum((key < cand).astype(jnp.int32), axis=1, keepdims=True)
        ans = jnp.where(c < KTH, cand, ans)
    fb = jnp.where(ans >= 0, ans, ans ^ _MAGM)
    thr_ref[...] = jax.lax.bitcast_convert_type(fb, jnp.float32).reshape(TG, 1, 1)


GB = 16


def _graph_kernel(thr_ref, v_ref, a_ref, Wi_ref, bi_ref, epsv_ref,
                  gW1_ref, s1_ref, t1_ref, gW2_ref, s2_ref, t2_ref,
                  sWe_ref, ses_ref, set_ref, sWa_ref, sba_ref,
                  hro_ref, og_ref):
    f32 = jnp.float32
    col = jax.lax.broadcasted_iota(jnp.int32, (N, N), 1)
    row = jax.lax.broadcasted_iota(jnp.int32, (N, N), 0)
    rs = range(GB)
    A = [(a_ref[g] > thr_ref[g, 0, 0]).astype(f32) for g in rs]
    x = [jax.lax.dot_general(v_ref[g], Wi_ref[...], (((1,), (1,)), ((), ())),
                             preferred_element_type=f32) + bi_ref[...]
         for g in rs]
    hros = [[] for _ in rs]
    ogs = [[] for _ in rs]
    for l in range(NL):
        epsb = epsv_ref[l:l + 1, :]
        s1 = s1_ref[l:l + 1, :]
        t1 = t1_ref[l:l + 1, :]
        s2 = s2_ref[l:l + 1, :]
        t2 = t2_ref[l:l + 1, :]
        agg = [jnp.dot(A[g], x[g], preferred_element_type=f32) + epsb * x[g]
               for g in rs]
        h1 = [jax.lax.dot_general(agg[g], gW1_ref[l], (((1,), (1,)), ((), ())),
                                  preferred_element_type=f32) for g in rs]
        z = [jax.nn.relu(h1[g] * s1 + t1) for g in rs]
        h2 = [jax.lax.dot_general(z[g], gW2_ref[l], (((1,), (1,)), ((), ())),
                                  preferred_element_type=f32) for g in rs]
        x = [jax.nn.relu(h2[g] * s2 + t2) for g in rs]
        xr = [jnp.mean(x[g], axis=0, keepdims=True) for g in rs]
        e = [jax.lax.dot_general(xr[g], sWe_ref[l], (((1,), (1,)), ((), ())),
                                 preferred_element_type=f32) for g in rs]
        ep = [e[g] * ses_ref[l:l + 1, :] + set_ref[l:l + 1, :] for g in rs]
        xe = [0.5 * ep[g] * (1.0 + jax.lax.erf(
            ep[g] * np.float32(1.0 / np.sqrt(2.0)))) for g in rs]
        gp = [jax.lax.dot_general(xe[g], sWa_ref[l], (((1,), (1,)), ((), ())),
                                  preferred_element_type=f32)
              + sba_ref[l:l + 1, :] for g in rs]
        ga = [jax.nn.sigmoid(gp[g]) for g in rs]
        for g in rs:
            hro = jnp.dot(ga[g], x[g], preferred_element_type=f32) \
                * np.float32(1.0 / N)
            hros[g].append(hro.reshape(1, 1, 1, HID))
        Mi = [jax.lax.dot_general(x[g], x[g], (((1,), (1,)), ((), ())),
                                  preferred_element_type=f32) for g in rs]
        for g in rs:
            rmax = jnp.max(Mi[g], axis=1, keepdims=True)
            Dn = Mi[g] / rmax
            D = jnp.where(col >= row, Dn - jnp.where(col == row, 1.0, 0.0), 0.0)
            ogs[g].append(jnp.sqrt(jnp.sum(D * D)).reshape(1, 1, 1))
    hro_ref[...] = jnp.concatenate(
        [jnp.concatenate(hros[g], axis=2) for g in rs], axis=1)
    og_ref[...] = jnp.concatenate(
        [jnp.concatenate(ogs[g], axis=2) for g in rs], axis=0)


def _ln(z, g, b):
    mu = jnp.mean(z, axis=-1, keepdims=True)
    var = jnp.mean((z - mu) ** 2, axis=-1, keepdims=True)
    return (z - mu) * jax.lax.rsqrt(var + 1e-5) * g + b


def _tx_kernel(h_ref, og_ref, tWin_ref, tbin_ref, tWout_ref, tbout_ref,
               ln1g_ref, ln1b_ref, ln2g_ref, ln2b_ref,
               mW1_ref, mb1_ref, mW2_ref, mb2_ref,
               lat_ref, reg_ref):
    f32 = jnp.float32
    DH = HID // NH
    lane_head = jax.lax.broadcasted_iota(jnp.int32, (1, HID), 1) // DH
    head_masks = [(lane_head == h).astype(f32) for h in range(NH)]
    lats = []
    for l in range(NL):
        h2 = h_ref[:, :, l, :].reshape(B * T, HID)
        qkv = jax.lax.dot_general(h2, tWin_ref[l], (((1,), (1,)), ((), ())),
                                  preferred_element_type=f32) + tbin_ref[l]
        q = qkv[:, :HID] * np.float32(1.0 / np.sqrt(DH))
        k = qkv[:, HID:2 * HID]
        v = qkv[:, 2 * HID:]
        qm = [q * head_masks[h] for h in range(NH)]
        vm = [v * head_masks[h] for h in range(NH)]
        bh = [(b, h) for b in range(B) for h in range(NH)]
        s = [jax.lax.dot_general(qm[h][b * T:(b + 1) * T, :],
                                 k[b * T:(b + 1) * T, :],
                                 (((1,), (1,)), ((), ())),
                                 preferred_element_type=f32) for b, h in bh]
        p = [jax.nn.softmax(sj, axis=-1) for sj in s]
        oc = [jnp.dot(p[j], vm[h][b * T:(b + 1) * T, :],
                      preferred_element_type=f32)
              for j, (b, h) in enumerate(bh)]
        obs = []
        for b in range(B):
            ob = oc[b * NH]
            for h in range(1, NH):
                ob = ob + oc[b * NH + h]
            obs.append(ob)
        o3 = jnp.concatenate(obs, axis=0)
        o = jax.lax.dot_general(o3, tWout_ref[l], (((1,), (1,)), ((), ())),
                                preferred_element_type=f32) + tbout_ref[l]
        xat = _ln(o, ln1g_ref[l], ln1b_ref[l])
        m1 = jax.nn.relu(
            jax.lax.dot_general(xat, mW1_ref[l], (((1,), (1,)), ((), ())),
                                preferred_element_type=f32) + mb1_ref[l])
        m2 = jax.lax.dot_general(m1, mW2_ref[l], (((1,), (1,)), ((), ())),
                                 preferred_element_type=f32) + mb2_ref[l]
        xat = _ln(xat + m2, ln2g_ref[l], ln2b_ref[l])
        lats.append(jnp.sum(xat.reshape(B, T, HID), axis=1))
    lat_ref[...] = jnp.concatenate(lats, axis=1)
    reg_ref[...] = (jnp.sum(og_ref[...]) * np.float32(1.0 / G)).reshape(1, 1)


def _full(shape):
    nd = len(shape)
    return pl.BlockSpec(shape, lambda i, _nd=nd: (0,) * _nd)


@jax.jit
def kernel(v1, a1, t1, sampling_endpoints1, W_init, b_init, eps,
           gW1, gb1, gW2, gb2,
           gbn1_g, gbn1_b, gbn1_rm, gbn1_rv,
           gbn2_g, gbn2_b, gbn2_rm, gbn2_rv,
           sbn_g, sbn_b, sbn_rm, sbn_rv,
           sW_e, sb_e, sW_a, sb_a,
           tWin, tbin, tWout, tbout,
           ln1_g, ln1_b, ln2_g, ln2_b,
           mW1, mb1, mW2, mb2):
    f32 = jnp.float32
    a_flat = a1.reshape(G, E)
    thr = pl.pallas_call(
        _thr_kernel,
        grid=(G // TG,),
        compiler_params=pltpu.CompilerParams(
            dimension_semantics=("parallel",)),
        in_specs=[pl.BlockSpec((TG, E), lambda i: (i, 0))],
        out_specs=pl.BlockSpec((TG, 1, 1), lambda i: (i, 0, 0)),
        out_shape=jax.ShapeDtypeStruct((G, 1, 1), f32),
    )(a_flat)

    def fold(bias, g, b, rm, rv):
        sc = g * jax.lax.rsqrt(rv + 1e-5)
        return sc, (bias - rm) * sc + b

    s1, t1f = fold(gb1, gbn1_g, gbn1_b, gbn1_rm, gbn1_rv)
    s2, t2f = fold(gb2, gbn2_g, gbn2_b, gbn2_rm, gbn2_rv)
    ses, setf = fold(sb_e, sbn_g, sbn_b, sbn_rm, sbn_rv)
    epsv = jnp.broadcast_to(eps.reshape(NL, 1), (NL, HID))

    thr3 = thr
    v3 = v1.reshape(G, N, CIN)
    a3 = a1.reshape(G, N, N)
    hro, og = pl.pallas_call(
        _graph_kernel,
        grid=(G // GB,),
        compiler_params=pltpu.CompilerParams(
            dimension_semantics=("parallel",)),
        in_specs=[
            pl.BlockSpec((GB, 1, 1), lambda i: (i, 0, 0)),
            pl.BlockSpec((GB, N, CIN), lambda i: (i, 0, 0)),
            pl.BlockSpec((GB, N, N), lambda i: (i, 0, 0)),
            _full((HID, CIN)), _full((1, HID)), _full((NL, HID)),
            _full((NL, HID, HID)), _full((NL, HID)), _full((NL, HID)),
            _full((NL, HID, HID)), _full((NL, HID)), _full((NL, HID)),
            _full((NL, HID, HID)), _full((NL, HID)), _full((NL, HID)),
            _full((NL, N, HID)), _full((NL, N)),
        ],
        out_specs=[
            pl.BlockSpec((1, GB, NL, HID), lambda i: (i // (T // GB), i % (T // GB), 0, 0)),
            pl.BlockSpec((GB, 1, NL), lambda i: (i, 0, 0)),
        ],
        out_shape=[
            jax.ShapeDtypeStruct((B, T, NL, HID), f32),
            jax.ShapeDtypeStruct((G, 1, NL), f32),
        ],
    )(thr3, v3, a3, W_init, b_init.reshape(1, HID), epsv,
      gW1, s1, t1f, gW2, s2, t2f, sW_e, ses, setf, sW_a, sb_a)

    lat, reg2 = pl.pallas_call(
        _tx_kernel,
        grid=(1,),
        in_specs=[
            _full((B, T, NL, HID)),
            _full((G, 1, NL)),
            _full((NL, 3 * HID, HID)),
            _full((NL, 1, 3 * HID)),
            _full((NL, HID, HID)),
            _full((NL, 1, HID)),
            _full((NL, 1, HID)),
            _full((NL, 1, HID)),
            _full((NL, 1, HID)),
            _full((NL, 1, HID)),
            _full((NL, 2 * HID, HID)),
            _full((NL, 1, 2 * HID)),
            _full((NL, HID, 2 * HID)),
            _full((NL, 1, HID)),
        ],
        out_specs=[
            pl.BlockSpec((B, NL * HID), lambda i: (0, 0)),
            pl.BlockSpec((1, 1), lambda i: (0, 0)),
        ],
        out_shape=[
            jax.ShapeDtypeStruct((B, NL * HID), f32),
            jax.ShapeDtypeStruct((1, 1), f32),
        ],
    )(hro, og, tWin, tbin.reshape(NL, 1, 3 * HID), tWout,
      tbout.reshape(NL, 1, HID),
      ln1_g.reshape(NL, 1, HID), ln1_b.reshape(NL, 1, HID),
      ln2_g.reshape(NL, 1, HID), ln2_b.reshape(NL, 1, HID),
      mW1, mb1.reshape(NL, 1, 2 * HID), mW2, mb2.reshape(NL, 1, HID))

    reg = reg2.reshape(())
    return lat, reg
